# baseline (speedup 1.0000x reference)
import jax
import jax.numpy as jnp
from jax.experimental import pallas as pl
from jax.experimental.pallas import tpu as pltpu

_T, _N, _D, _M = 2, 100, 128, 1024
_R = _T * _N
_MARGIN = 0.5


def _l2n(x):
    return x / jnp.maximum(jnp.sqrt(jnp.sum(x * x, axis=-1, keepdims=True)), 1e-12)


def _body(x_ref, th_ref, k_ref, v_ref,
          nemb_ref, eg_ref, lk_ref, lv_ref, ls_ref):
    ne = _l2n(x_ref[0:_R, :])
    ng = _l2n(x_ref[_R:2 * _R, :])
    nemb_ref[...] = ne

    kmat = k_ref[...]
    vmat = v_ref[...]
    sim_kv = jax.lax.dot_general(ne, kmat, (((1,), (1,)), ((), ())),
                                 preferred_element_type=jnp.float32)
    sim_vk = jax.lax.dot_general(ng, vmat, (((1,), (1,)), ((), ())),
                                 preferred_element_type=jnp.float32)

    th0 = th_ref[0]
    th1 = th_ref[1]
    th2 = th_ref[2]
    th3 = th_ref[3]

    pos_score = jnp.where(sim_kv > th0, sim_kv, 0.0)
    eg = ng + jax.lax.dot_general(pos_score, vmat, (((1,), (0,)), ((), ())),
                                  preferred_element_type=jnp.float32)
    eg_ref[...] = _l2n(eg)

    diff = sim_vk - sim_kv
    ls_ref[0] = jnp.sum(diff * diff) / (_R * _M)

    iota = jax.lax.broadcasted_iota(jnp.int32, (_R, _M), 1)
    big = jnp.int32(2 ** 30)
    inf = jnp.float32(jnp.inf)

    def pair_contrib(src, other, thp, thn):
        mp = jnp.where(src > thp, src, inf)
        mn = jnp.where(src < thn, src, -inf)
        extp = jnp.min(mp, axis=1, keepdims=True)
        extn = jnp.max(mn, axis=1, keepdims=True)
        idxp = jnp.min(jnp.where(mp == extp, iota, big), axis=1, keepdims=True)
        idxn = jnp.min(jnp.where(mn == extn, iota, big), axis=1, keepdims=True)
        valp = jnp.sum(jnp.where(iota == idxp, other, 0.0), axis=1, keepdims=True)
        valn = jnp.sum(jnp.where(iota == idxn, other, 0.0), axis=1, keepdims=True)
        anyp = (extp != inf).astype(jnp.float32)
        anyn = (extn != -inf).astype(jnp.float32)
        return jnp.sum(anyp * valp - anyn * valn)

    lv_ref[0] = jnp.maximum(
        -pair_contrib(sim_kv, sim_vk, th0, th1) / _R + _MARGIN, 0.0)
    lk_ref[0] = jnp.maximum(
        -pair_contrib(sim_vk, sim_kv, th2, th3) / _R + _MARGIN, 0.0)


def kernel(emb_support, emb_query, glo_support, glo_query, thresh,
           memory_keys, memory_values):
    x = jnp.concatenate(
        [emb_support[0], emb_query[0], emb_support[1], emb_query[1],
         glo_support[0], glo_query[0], glo_support[1], glo_query[1]],
        axis=0)

    out_shape = (
        jax.ShapeDtypeStruct((_R, _D), jnp.float32),
        jax.ShapeDtypeStruct((_R, _D), jnp.float32),
        jax.ShapeDtypeStruct((1,), jnp.float32),
        jax.ShapeDtypeStruct((1,), jnp.float32),
        jax.ShapeDtypeStruct((1,), jnp.float32),
    )
    vspec = pl.BlockSpec(memory_space=pltpu.VMEM)
    sspec = pl.BlockSpec(memory_space=pltpu.SMEM)
    in_specs = [vspec, sspec, vspec, vspec]
    out_specs = (vspec, vspec, sspec, sspec, sspec)
    ne, eg, lk, lv, ls = pl.pallas_call(
        _body,
        out_shape=out_shape,
        in_specs=in_specs,
        out_specs=out_specs,
    )(x, thresh, memory_keys, memory_values)

    return (ne.reshape(_T, _N, _D), eg.reshape(_T, _N, _D),
            lk[0], lv[0], ls[0])
